# baseline (speedup 1.0000x reference)
import jax
import jax.numpy as jnp
from jax.experimental import pallas as pl

_BLOCK = 1024
_CHAINS = 4
_SUB = _BLOCK // _CHAINS


def _split3cat(e):
    hi = e.astype(jnp.bfloat16)
    r1 = e - hi.astype(jnp.float32)
    mid = r1.astype(jnp.bfloat16)
    lo = (r1 - mid.astype(jnp.float32)).astype(jnp.bfloat16)
    return jnp.concatenate([hi, mid, lo], axis=1)


def _vq_body(z_ref, rn_ref, e0_ref, e1_ref, en0_ref, en1_ref,
             E0_ref, E1_ref, q_ref, i0_ref, i1_ref, l0_ref, l1_ref):
    k = e0_ref.shape[0]
    d = z_ref.shape[1]

    def layer(r, r2, rnorm, e_ref, en_ref, E_ref):
        s2 = jax.lax.dot_general(r2, e_ref[...], (((1,), (1,)), ((), ())),
                                 preferred_element_type=jnp.float32)
        en = en_ref[...]
        iota = jax.lax.broadcasted_iota(jnp.int32, s2.shape, 1)
        w = k // 2
        a = (rnorm + en[:, :w]) - s2[:, :w]
        b = (rnorm + en[:, w:]) - s2[:, w:]
        take_b = b < a
        v = jnp.minimum(a, b)
        i = jnp.where(take_b, iota[:, w:], iota[:, :w])
        while w > 128:
            w //= 2
            a, b = v[:, :w], v[:, w:]
            ia, ib = i[:, :w], i[:, w:]
            take_b = b < a
            v = jnp.minimum(a, b)
            i = jnp.where(take_b, ib, ia)
        m = jnp.min(v, axis=1, keepdims=True)
        idx = jnp.min(jnp.where(v == m, i, k), axis=1)
        oh = (iota == idx[:, None]).astype(jnp.bfloat16)
        g = jax.lax.dot_general(oh, E_ref[...], (((1,), (0,)), ((), ())),
                                preferred_element_type=jnp.float32)
        q = (g[:, :d] + g[:, d:2 * d]) + g[:, 2 * d:3 * d]
        t = q - r
        qs = r + t
        return qs, idx, jnp.sum(t * t).reshape(1, 1)

    lsum0 = jnp.zeros((1, 1), jnp.float32)
    lsum1 = jnp.zeros((1, 1), jnp.float32)
    for h in range(_CHAINS):
        sl = pl.ds(h * _SUB, _SUB)
        z = z_ref[sl, :]
        z2 = z + z
        rn = rn_ref[sl, :]
        qs0, idx0, l0 = layer(z, z2, rn, e0_ref, en0_ref, E0_ref)
        r1 = z - qs0
        r1_2 = r1 + r1
        rn1 = jnp.sum(r1 * r1, axis=1, keepdims=True)
        qs1, idx1, l1 = layer(r1, r1_2, rn1, e1_ref, en1_ref, E1_ref)
        q_ref[sl, :] = qs0 + qs1
        i0_ref[sl, :] = idx0[:, None]
        i1_ref[sl, :] = idx1[:, None]
        lsum0 += l0
        lsum1 += l1

    @pl.when(pl.program_id(0) == 0)
    def _():
        l0_ref[...] = jnp.zeros((1, 1), jnp.float32)
        l1_ref[...] = jnp.zeros((1, 1), jnp.float32)

    l0_ref[...] += lsum0
    l1_ref[...] += lsum1


def kernel(z_flat, codebook0, codebook1):
    n, d = z_flat.shape
    k = codebook0.shape[0]
    rn = jnp.sum(z_flat ** 2, axis=1, keepdims=True)
    en0 = jnp.sum(codebook0 ** 2, axis=1).reshape(1, k)
    en1 = jnp.sum(codebook1 ** 2, axis=1).reshape(1, k)
    E0 = _split3cat(codebook0)
    E1 = _split3cat(codebook1)

    row = lambda i: (i, 0)
    rep = lambda i: (0, 0)
    q, i0, i1, l0, l1 = pl.pallas_call(
        _vq_body,
        grid=(n // _BLOCK,),
        in_specs=[
            pl.BlockSpec((_BLOCK, d), row),
            pl.BlockSpec((_BLOCK, 1), row),
            pl.BlockSpec((k, d), rep),
            pl.BlockSpec((k, d), rep),
            pl.BlockSpec((1, k), rep),
            pl.BlockSpec((1, k), rep),
            pl.BlockSpec((k, 3 * d), rep),
            pl.BlockSpec((k, 3 * d), rep),
        ],
        out_specs=[
            pl.BlockSpec((_BLOCK, d), row),
            pl.BlockSpec((_BLOCK, 1), row),
            pl.BlockSpec((_BLOCK, 1), row),
            pl.BlockSpec((1, 1), rep),
            pl.BlockSpec((1, 1), rep),
        ],
        out_shape=[
            jax.ShapeDtypeStruct((n, d), jnp.float32),
            jax.ShapeDtypeStruct((n, 1), jnp.int32),
            jax.ShapeDtypeStruct((n, 1), jnp.int32),
            jax.ShapeDtypeStruct((1, 1), jnp.float32),
            jax.ShapeDtypeStruct((1, 1), jnp.float32),
        ],
    )(z_flat, rn, codebook0, codebook1, en0, en1, E0, E1)

    nd = jnp.float32(n * d)
    m0 = l0[0, 0] / nd
    m1 = l1[0, 0] / nd
    loss0 = m0 + 0.25 * m0
    loss1 = m1 + 0.25 * m1
    total = loss0 + loss1
    return (total, q, i0.reshape(n), i1.reshape(n))


# speedup vs baseline: 1.2023x; 1.2023x over previous
import jax
import jax.numpy as jnp
from jax.experimental import pallas as pl

_BLOCK = 2048
_CHAINS = 2
_SUB = _BLOCK // _CHAINS


def _split3cat(e):
    hi = e.astype(jnp.bfloat16)
    r1 = e - hi.astype(jnp.float32)
    mid = r1.astype(jnp.bfloat16)
    lo = (r1 - mid.astype(jnp.float32)).astype(jnp.bfloat16)
    return jnp.concatenate([hi, mid, lo], axis=1)


def _vq_body(z_ref, rn_ref, e0_ref, e1_ref, en0_ref, en1_ref,
             E0_ref, E1_ref, q_ref, i0_ref, i1_ref, l0_ref, l1_ref):
    k = e0_ref.shape[0]
    d = z_ref.shape[1]

    def layer(r, r2, rnorm, e_ref, en_ref, E_ref):
        s2 = jax.lax.dot_general(r2, e_ref[...], (((1,), (1,)), ((), ())),
                                 preferred_element_type=jnp.float32)
        en = en_ref[...]
        iota = jax.lax.broadcasted_iota(jnp.int32, s2.shape, 1)
        w = k // 2
        a = (rnorm + en[:, :w]) - s2[:, :w]
        b = (rnorm + en[:, w:]) - s2[:, w:]
        take_b = b < a
        v = jnp.minimum(a, b)
        i = jnp.where(take_b, iota[:, w:], iota[:, :w])
        while w > 128:
            w //= 2
            a, b = v[:, :w], v[:, w:]
            ia, ib = i[:, :w], i[:, w:]
            take_b = b < a
            v = jnp.minimum(a, b)
            i = jnp.where(take_b, ib, ia)
        m = jnp.min(v, axis=1, keepdims=True)
        idx = jnp.min(jnp.where(v == m, i, k), axis=1)
        oh = (iota == idx[:, None]).astype(jnp.bfloat16)
        g = jax.lax.dot_general(oh, E_ref[...], (((1,), (0,)), ((), ())),
                                preferred_element_type=jnp.float32)
        q = (g[:, :d] + g[:, d:2 * d]) + g[:, 2 * d:3 * d]
        t = q - r
        qs = r + t
        return qs, idx, jnp.sum(t * t).reshape(1, 1)

    lsum0 = jnp.zeros((1, 1), jnp.float32)
    lsum1 = jnp.zeros((1, 1), jnp.float32)
    for h in range(_CHAINS):
        sl = pl.ds(h * _SUB, _SUB)
        z = z_ref[sl, :]
        z2 = z + z
        rn = rn_ref[sl, :]
        qs0, idx0, l0 = layer(z, z2, rn, e0_ref, en0_ref, E0_ref)
        r1 = z - qs0
        r1_2 = r1 + r1
        rn1 = jnp.sum(r1 * r1, axis=1, keepdims=True)
        qs1, idx1, l1 = layer(r1, r1_2, rn1, e1_ref, en1_ref, E1_ref)
        q_ref[sl, :] = qs0 + qs1
        i0_ref[sl, :] = idx0[:, None]
        i1_ref[sl, :] = idx1[:, None]
        lsum0 += l0
        lsum1 += l1

    @pl.when(pl.program_id(0) == 0)
    def _():
        l0_ref[...] = jnp.zeros((1, 1), jnp.float32)
        l1_ref[...] = jnp.zeros((1, 1), jnp.float32)

    l0_ref[...] += lsum0
    l1_ref[...] += lsum1


def kernel(z_flat, codebook0, codebook1):
    n, d = z_flat.shape
    k = codebook0.shape[0]
    rn = jnp.sum(z_flat ** 2, axis=1, keepdims=True)
    en0 = jnp.sum(codebook0 ** 2, axis=1).reshape(1, k)
    en1 = jnp.sum(codebook1 ** 2, axis=1).reshape(1, k)
    E0 = _split3cat(codebook0)
    E1 = _split3cat(codebook1)

    row = lambda i: (i, 0)
    rep = lambda i: (0, 0)
    q, i0, i1, l0, l1 = pl.pallas_call(
        _vq_body,
        grid=(n // _BLOCK,),
        in_specs=[
            pl.BlockSpec((_BLOCK, d), row),
            pl.BlockSpec((_BLOCK, 1), row),
            pl.BlockSpec((k, d), rep),
            pl.BlockSpec((k, d), rep),
            pl.BlockSpec((1, k), rep),
            pl.BlockSpec((1, k), rep),
            pl.BlockSpec((k, 3 * d), rep),
            pl.BlockSpec((k, 3 * d), rep),
        ],
        out_specs=[
            pl.BlockSpec((_BLOCK, d), row),
            pl.BlockSpec((_BLOCK, 1), row),
            pl.BlockSpec((_BLOCK, 1), row),
            pl.BlockSpec((1, 1), rep),
            pl.BlockSpec((1, 1), rep),
        ],
        out_shape=[
            jax.ShapeDtypeStruct((n, d), jnp.float32),
            jax.ShapeDtypeStruct((n, 1), jnp.int32),
            jax.ShapeDtypeStruct((n, 1), jnp.int32),
            jax.ShapeDtypeStruct((1, 1), jnp.float32),
            jax.ShapeDtypeStruct((1, 1), jnp.float32),
        ],
    )(z_flat, rn, codebook0, codebook1, en0, en1, E0, E1)

    nd = jnp.float32(n * d)
    m0 = l0[0, 0] / nd
    m1 = l1[0, 0] / nd
    loss0 = m0 + 0.25 * m0
    loss1 = m1 + 0.25 * m1
    total = loss0 + loss1
    return (total, q, i0.reshape(n), i1.reshape(n))


# speedup vs baseline: 1.2224x; 1.0168x over previous
import jax
import jax.numpy as jnp
from jax.experimental import pallas as pl

_BLOCK = 4096
_CHAINS = 2
_SUB = _BLOCK // _CHAINS


def _split3cat(e):
    hi = e.astype(jnp.bfloat16)
    r1 = e - hi.astype(jnp.float32)
    mid = r1.astype(jnp.bfloat16)
    lo = (r1 - mid.astype(jnp.float32)).astype(jnp.bfloat16)
    return jnp.concatenate([hi, mid, lo], axis=1)


def _vq_body(z_ref, rn_ref, e0_ref, e1_ref, en0_ref, en1_ref,
             E0_ref, E1_ref, q_ref, i0_ref, i1_ref, l0_ref, l1_ref):
    k = e0_ref.shape[0]
    d = z_ref.shape[1]

    def layer(r, r2, rnorm, e_ref, en_ref, E_ref):
        s2 = jax.lax.dot_general(r2, e_ref[...], (((1,), (1,)), ((), ())),
                                 preferred_element_type=jnp.float32)
        en = en_ref[...]
        iota = jax.lax.broadcasted_iota(jnp.int32, s2.shape, 1)
        w = k // 2
        a = (rnorm + en[:, :w]) - s2[:, :w]
        b = (rnorm + en[:, w:]) - s2[:, w:]
        take_b = b < a
        v = jnp.minimum(a, b)
        i = jnp.where(take_b, iota[:, w:], iota[:, :w])
        while w > 128:
            w //= 2
            a, b = v[:, :w], v[:, w:]
            ia, ib = i[:, :w], i[:, w:]
            take_b = b < a
            v = jnp.minimum(a, b)
            i = jnp.where(take_b, ib, ia)
        m = jnp.min(v, axis=1, keepdims=True)
        idx = jnp.min(jnp.where(v == m, i, k), axis=1)
        oh = (iota == idx[:, None]).astype(jnp.bfloat16)
        g = jax.lax.dot_general(oh, E_ref[...], (((1,), (0,)), ((), ())),
                                preferred_element_type=jnp.float32)
        q = (g[:, :d] + g[:, d:2 * d]) + g[:, 2 * d:3 * d]
        t = q - r
        qs = r + t
        return qs, idx, jnp.sum(t * t).reshape(1, 1)

    lsum0 = jnp.zeros((1, 1), jnp.float32)
    lsum1 = jnp.zeros((1, 1), jnp.float32)
    for h in range(_CHAINS):
        sl = pl.ds(h * _SUB, _SUB)
        z = z_ref[sl, :]
        z2 = z + z
        rn = rn_ref[sl, :]
        qs0, idx0, l0 = layer(z, z2, rn, e0_ref, en0_ref, E0_ref)
        r1 = z - qs0
        r1_2 = r1 + r1
        rn1 = jnp.sum(r1 * r1, axis=1, keepdims=True)
        qs1, idx1, l1 = layer(r1, r1_2, rn1, e1_ref, en1_ref, E1_ref)
        q_ref[sl, :] = qs0 + qs1
        i0_ref[sl, :] = idx0[:, None]
        i1_ref[sl, :] = idx1[:, None]
        lsum0 += l0
        lsum1 += l1

    @pl.when(pl.program_id(0) == 0)
    def _():
        l0_ref[...] = jnp.zeros((1, 1), jnp.float32)
        l1_ref[...] = jnp.zeros((1, 1), jnp.float32)

    l0_ref[...] += lsum0
    l1_ref[...] += lsum1


def kernel(z_flat, codebook0, codebook1):
    n, d = z_flat.shape
    k = codebook0.shape[0]
    rn = jnp.sum(z_flat ** 2, axis=1, keepdims=True)
    en0 = jnp.sum(codebook0 ** 2, axis=1).reshape(1, k)
    en1 = jnp.sum(codebook1 ** 2, axis=1).reshape(1, k)
    E0 = _split3cat(codebook0)
    E1 = _split3cat(codebook1)

    row = lambda i: (i, 0)
    rep = lambda i: (0, 0)
    q, i0, i1, l0, l1 = pl.pallas_call(
        _vq_body,
        grid=(n // _BLOCK,),
        in_specs=[
            pl.BlockSpec((_BLOCK, d), row),
            pl.BlockSpec((_BLOCK, 1), row),
            pl.BlockSpec((k, d), rep),
            pl.BlockSpec((k, d), rep),
            pl.BlockSpec((1, k), rep),
            pl.BlockSpec((1, k), rep),
            pl.BlockSpec((k, 3 * d), rep),
            pl.BlockSpec((k, 3 * d), rep),
        ],
        out_specs=[
            pl.BlockSpec((_BLOCK, d), row),
            pl.BlockSpec((_BLOCK, 1), row),
            pl.BlockSpec((_BLOCK, 1), row),
            pl.BlockSpec((1, 1), rep),
            pl.BlockSpec((1, 1), rep),
        ],
        out_shape=[
            jax.ShapeDtypeStruct((n, d), jnp.float32),
            jax.ShapeDtypeStruct((n, 1), jnp.int32),
            jax.ShapeDtypeStruct((n, 1), jnp.int32),
            jax.ShapeDtypeStruct((1, 1), jnp.float32),
            jax.ShapeDtypeStruct((1, 1), jnp.float32),
        ],
    )(z_flat, rn, codebook0, codebook1, en0, en1, E0, E1)

    nd = jnp.float32(n * d)
    m0 = l0[0, 0] / nd
    m1 = l1[0, 0] / nd
    loss0 = m0 + 0.25 * m0
    loss1 = m1 + 0.25 * m1
    total = loss0 + loss1
    return (total, q, i0.reshape(n), i1.reshape(n))


# speedup vs baseline: 1.2392x; 1.0137x over previous
import jax
import jax.numpy as jnp
from jax.experimental import pallas as pl

_BLOCK = 6144
_CHAINS = 3
_SUB = _BLOCK // _CHAINS


def _split3cat(e):
    hi = e.astype(jnp.bfloat16)
    r1 = e - hi.astype(jnp.float32)
    mid = r1.astype(jnp.bfloat16)
    lo = (r1 - mid.astype(jnp.float32)).astype(jnp.bfloat16)
    return jnp.concatenate([hi, mid, lo], axis=1)


def _vq_body(z_ref, rn_ref, e0_ref, e1_ref, en0_ref, en1_ref,
             E0_ref, E1_ref, q_ref, i0_ref, i1_ref, l0_ref, l1_ref):
    k = e0_ref.shape[0]
    d = z_ref.shape[1]

    def layer(r, r2, rnorm, e_ref, en_ref, E_ref):
        s2 = jax.lax.dot_general(r2, e_ref[...], (((1,), (1,)), ((), ())),
                                 preferred_element_type=jnp.float32)
        en = en_ref[...]
        iota = jax.lax.broadcasted_iota(jnp.int32, s2.shape, 1)
        w = k // 2
        a = (rnorm + en[:, :w]) - s2[:, :w]
        b = (rnorm + en[:, w:]) - s2[:, w:]
        take_b = b < a
        v = jnp.minimum(a, b)
        i = jnp.where(take_b, iota[:, w:], iota[:, :w])
        while w > 128:
            w //= 2
            a, b = v[:, :w], v[:, w:]
            ia, ib = i[:, :w], i[:, w:]
            take_b = b < a
            v = jnp.minimum(a, b)
            i = jnp.where(take_b, ib, ia)
        m = jnp.min(v, axis=1, keepdims=True)
        idx = jnp.min(jnp.where(v == m, i, k), axis=1)
        oh = (iota == idx[:, None]).astype(jnp.bfloat16)
        g = jax.lax.dot_general(oh, E_ref[...], (((1,), (0,)), ((), ())),
                                preferred_element_type=jnp.float32)
        q = (g[:, :d] + g[:, d:2 * d]) + g[:, 2 * d:3 * d]
        t = q - r
        qs = r + t
        return qs, idx, jnp.sum(t * t).reshape(1, 1)

    lsum0 = jnp.zeros((1, 1), jnp.float32)
    lsum1 = jnp.zeros((1, 1), jnp.float32)
    for h in range(_CHAINS):
        sl = pl.ds(h * _SUB, _SUB)
        z = z_ref[sl, :]
        z2 = z + z
        rn = rn_ref[sl, :]
        qs0, idx0, l0 = layer(z, z2, rn, e0_ref, en0_ref, E0_ref)
        r1 = z - qs0
        r1_2 = r1 + r1
        rn1 = jnp.sum(r1 * r1, axis=1, keepdims=True)
        qs1, idx1, l1 = layer(r1, r1_2, rn1, e1_ref, en1_ref, E1_ref)
        q_ref[sl, :] = qs0 + qs1
        i0_ref[sl, :] = idx0[:, None]
        i1_ref[sl, :] = idx1[:, None]
        lsum0 += l0
        lsum1 += l1

    @pl.when(pl.program_id(0) == 0)
    def _():
        l0_ref[...] = jnp.zeros((1, 1), jnp.float32)
        l1_ref[...] = jnp.zeros((1, 1), jnp.float32)

    l0_ref[...] += lsum0
    l1_ref[...] += lsum1


def kernel(z_flat, codebook0, codebook1):
    n, d = z_flat.shape
    k = codebook0.shape[0]
    rn = jnp.sum(z_flat ** 2, axis=1, keepdims=True)
    en0 = jnp.sum(codebook0 ** 2, axis=1).reshape(1, k)
    en1 = jnp.sum(codebook1 ** 2, axis=1).reshape(1, k)
    E0 = _split3cat(codebook0)
    E1 = _split3cat(codebook1)

    row = lambda i: (i, 0)
    rep = lambda i: (0, 0)
    q, i0, i1, l0, l1 = pl.pallas_call(
        _vq_body,
        grid=(n // _BLOCK,),
        in_specs=[
            pl.BlockSpec((_BLOCK, d), row),
            pl.BlockSpec((_BLOCK, 1), row),
            pl.BlockSpec((k, d), rep),
            pl.BlockSpec((k, d), rep),
            pl.BlockSpec((1, k), rep),
            pl.BlockSpec((1, k), rep),
            pl.BlockSpec((k, 3 * d), rep),
            pl.BlockSpec((k, 3 * d), rep),
        ],
        out_specs=[
            pl.BlockSpec((_BLOCK, d), row),
            pl.BlockSpec((_BLOCK, 1), row),
            pl.BlockSpec((_BLOCK, 1), row),
            pl.BlockSpec((1, 1), rep),
            pl.BlockSpec((1, 1), rep),
        ],
        out_shape=[
            jax.ShapeDtypeStruct((n, d), jnp.float32),
            jax.ShapeDtypeStruct((n, 1), jnp.int32),
            jax.ShapeDtypeStruct((n, 1), jnp.int32),
            jax.ShapeDtypeStruct((1, 1), jnp.float32),
            jax.ShapeDtypeStruct((1, 1), jnp.float32),
        ],
    )(z_flat, rn, codebook0, codebook1, en0, en1, E0, E1)

    nd = jnp.float32(n * d)
    m0 = l0[0, 0] / nd
    m1 = l1[0, 0] / nd
    loss0 = m0 + 0.25 * m0
    loss1 = m1 + 0.25 * m1
    total = loss0 + loss1
    return (total, q, i0.reshape(n), i1.reshape(n))
